# baseline (speedup 1.0000x reference)
import functools

import jax
import jax.numpy as jnp
from jax import lax
from jax.experimental import pallas as pl
from jax.experimental.pallas import tpu as pltpu
from jax.experimental.pallas import tpu_sc as plsc

B, S = 4, 8192
D = 1024
NT = 8
N = B * S
NC, NS = 2, 16
NW = NC * NS
TOK_PER_W = N // NW
C = 16
NCHUNK = TOK_PER_W // C
OUTBYTES = C * D * 4

_mesh = plsc.VectorSubcoreMesh(core_axis_name="c", subcore_axis_name="s")


@functools.partial(
    pl.kernel,
    mesh=_mesh,
    compiler_params=pltpu.CompilerParams(needs_layout_passes=False),
    out_type=jax.ShapeDtypeStruct((N, D), jnp.float32),
    scratch_types=[
        pltpu.VMEM((TOK_PER_W + C,), jnp.int32),
        pltpu.VMEM((TOK_PER_W,), jnp.int32),
        pltpu.VMEM((NT * D,), jnp.float32),
        pltpu.VMEM((C, D), jnp.float32),
        pltpu.VMEM((C, D), jnp.float32),
        pltpu.SemaphoreType.DMA,
        pltpu.SemaphoreType.DMA,
        pltpu.SemaphoreType.DMA,
        pltpu.SemaphoreType.DMA,
    ],
)
def _emb(ids_hbm, cids_hbm, tok_tbl_hbm, code_tbl_hbm, out_hbm,
         idx_all, cidx_all, ctbl_v, tok0, tok1,
         gsem0, gsem1, osem0, osem1):
    wid = lax.axis_index("s") * NC + lax.axis_index("c")
    base = wid * TOK_PER_W
    toks = (tok0, tok1)
    gsems = (gsem0, gsem1)
    osems = (osem0, osem1)

    pltpu.sync_copy(ids_hbm.at[pl.ds(base, TOK_PER_W)],
                    idx_all.at[pl.ds(0, TOK_PER_W)])
    pltpu.sync_copy(cids_hbm.at[pl.ds(base, TOK_PER_W)], cidx_all)
    pltpu.sync_copy(code_tbl_hbm, ctbl_v)
    for p in range(C // 16):
        idx_all[pl.ds(TOK_PER_W + p * 16, 16)] = jnp.zeros((16,), jnp.int32)

    iota16 = lax.iota(jnp.int32, 16)
    zeros16 = jnp.zeros((16,), jnp.int32)

    def start_gather(c, b):
        pltpu.async_copy(tok_tbl_hbm.at[idx_all.at[pl.ds(c * C, C)]],
                         toks[b], gsems[b])

    def wait_gather(b):
        pltpu.make_async_copy(tok_tbl_hbm.at[pl.ds(0, C)], toks[b],
                              gsems[b]).wait()

    def wait_out(b):
        pltpu.make_async_copy(toks[b], out_hbm.at[pl.ds(0, C)],
                              osems[b]).wait()

    NSEG = D // 16
    DEPTH = 8

    def compute(c, b):
        tok_v = toks[b]
        for g in range(C // 16):
            cvec = plsc.load_gather(
                cidx_all, [zeros16 + (c * C + g * 16) + iota16])
            for rl in range(16):
                r = g * 16 + rl
                spl_r = jnp.full((16,), r, jnp.int32)
                cid_s = lax.reduce_max(jnp.where(iota16 == rl, cvec, 0), (0,))
                cbase = lax.shift_left(zeros16 + cid_s, 10)
                cols = [iota16 + j * 16 for j in range(NSEG)]
                ring = [plsc.load_gather(ctbl_v, [cbase + cols[j]])
                        for j in range(DEPTH)]
                for j in range(DEPTH, NSEG):
                    plsc.addupdate_scatter(tok_v, [spl_r, cols[j - DEPTH]],
                                           ring[j % DEPTH])
                    ring[j % DEPTH] = plsc.load_gather(
                        ctbl_v, [cbase + cols[j]])
                for j in range(NSEG - DEPTH, NSEG):
                    plsc.addupdate_scatter(tok_v, [spl_r, cols[j]],
                                           ring[j % DEPTH])

    start_gather(0, 0)

    def _pair(i, carry):
        @pl.when(i > 0)
        def _():
            wait_out(1)

        start_gather(2 * i + 1, 1)
        wait_gather(0)
        compute(2 * i, 0)
        pltpu.async_copy(toks[0], out_hbm.at[pl.ds(base + 2 * i * C, C)],
                         osems[0])

        wait_out(0)
        start_gather(2 * i + 2, 0)
        wait_gather(1)
        compute(2 * i + 1, 1)
        pltpu.async_copy(toks[1],
                         out_hbm.at[pl.ds(base + (2 * i + 1) * C, C)],
                         osems[1])
        return carry

    lax.fori_loop(0, NCHUNK // 2, _pair, 0)

    wait_gather(0)
    wait_out(1)


def kernel(input_ids, code_type_ids, token_table, code_table):
    ids = input_ids.reshape(N).astype(jnp.int32)
    cids = code_type_ids.reshape(N).astype(jnp.int32)
    out = _emb(ids, cids, token_table, code_table.reshape(NT * D))
    return out.reshape(B, S, D)


# speedup vs baseline: 1.2867x; 1.2867x over previous
import functools

import jax
import jax.numpy as jnp
from jax import lax
from jax.experimental import pallas as pl
from jax.experimental.pallas import tpu as pltpu
from jax.experimental.pallas import tpu_sc as plsc

B, S = 4, 8192
D = 1024
NT = 8
N = B * S
NC, NS = 2, 16
NW = NC * NS
TOK_PER_W = N // NW
C = 32
NCHUNK = TOK_PER_W // C

_mesh = plsc.VectorSubcoreMesh(core_axis_name="c", subcore_axis_name="s")


@functools.partial(
    pl.kernel,
    mesh=_mesh,
    compiler_params=pltpu.CompilerParams(needs_layout_passes=False),
    out_type=jax.ShapeDtypeStruct((N, D), jnp.float32),
    scratch_types=[
        pltpu.VMEM((TOK_PER_W + C,), jnp.int32),
        pltpu.VMEM((TOK_PER_W,), jnp.int32),
        pltpu.VMEM((NT * D,), jnp.float32),
        pltpu.VMEM((2 * C, D), jnp.float32),
        pltpu.SemaphoreType.DMA,
        pltpu.SemaphoreType.DMA,
        pltpu.SemaphoreType.DMA,
        pltpu.SemaphoreType.DMA,
    ],
)
def _emb(ids_hbm, cids_hbm, tok_tbl_hbm, code_tbl_hbm, out_hbm,
         idx_all, cidx_all, ctbl_v, tok_all,
         gsem0, gsem1, osem0, osem1):
    wid = lax.axis_index("s") * NC + lax.axis_index("c")
    base = wid * TOK_PER_W
    gsems = (gsem0, gsem1)
    osems = (osem0, osem1)

    pltpu.sync_copy(ids_hbm.at[pl.ds(base, TOK_PER_W)],
                    idx_all.at[pl.ds(0, TOK_PER_W)])
    pltpu.sync_copy(cids_hbm.at[pl.ds(base, TOK_PER_W)], cidx_all)
    pltpu.sync_copy(code_tbl_hbm, ctbl_v)
    for p in range(C // 16):
        idx_all[pl.ds(TOK_PER_W + p * 16, 16)] = jnp.zeros((16,), jnp.int32)

    iota16 = lax.iota(jnp.int32, 16)
    zeros16 = jnp.zeros((16,), jnp.int32)

    def start_gather(c, b):
        pltpu.async_copy(tok_tbl_hbm.at[idx_all.at[pl.ds(c * C, C)]],
                         tok_all.at[pl.ds(b * C, C)], gsems[b])

    def wait_gather(b):
        pltpu.make_async_copy(tok_tbl_hbm.at[pl.ds(0, C)],
                              tok_all.at[pl.ds(0, C)], gsems[b]).wait()

    def start_out(c, b):
        pltpu.async_copy(tok_all.at[pl.ds(b * C, C)],
                         out_hbm.at[pl.ds(base + c * C, C)], osems[b])

    def wait_out(b):
        pltpu.make_async_copy(tok_all.at[pl.ds(0, C)],
                              out_hbm.at[pl.ds(0, C)], osems[b]).wait()

    NSEG = D // 16
    DEPTH = 8

    def compute(c, off_vec):
        for g in range(C // 16):
            cvec = plsc.load_gather(
                cidx_all, [zeros16 + (c * C + g * 16) + iota16])
            for rl in range(16):
                r = g * 16 + rl
                spl_r = off_vec + r
                cid_s = lax.reduce_max(jnp.where(iota16 == rl, cvec, 0), (0,))
                cbase = lax.shift_left(zeros16 + cid_s, 10)
                cols = [iota16 + j * 16 for j in range(NSEG)]
                ring = [plsc.load_gather(ctbl_v, [cbase + cols[j]])
                        for j in range(DEPTH)]
                for j in range(DEPTH, NSEG):
                    plsc.addupdate_scatter(tok_all, [spl_r, cols[j - DEPTH]],
                                           ring[j % DEPTH])
                    ring[j % DEPTH] = plsc.load_gather(
                        ctbl_v, [cbase + cols[j]])
                for j in range(NSEG - DEPTH, NSEG):
                    plsc.addupdate_scatter(tok_all, [spl_r, cols[j]],
                                           ring[j % DEPTH])

    start_gather(0, 0)

    def _phase(c, carry):
        par = lax.rem(c, 2)
        even = par == 0

        @pl.when(even & (c > 0))
        def _():
            wait_out(1)

        @pl.when(jnp.logical_not(even))
        def _():
            wait_out(0)

        @pl.when(even)
        def _():
            start_gather(c + 1, 1)
            wait_gather(0)

        @pl.when(jnp.logical_not(even))
        def _():
            start_gather(c + 1, 0)
            wait_gather(1)

        compute(c, zeros16 + par * C)

        @pl.when(even)
        def _():
            start_out(c, 0)

        @pl.when(jnp.logical_not(even))
        def _():
            start_out(c, 1)

        return carry

    lax.fori_loop(0, NCHUNK, _phase, 0)

    wait_gather(0)
    wait_out(1)


def kernel(input_ids, code_type_ids, token_table, code_table):
    ids = input_ids.reshape(N).astype(jnp.int32)
    cids = code_type_ids.reshape(N).astype(jnp.int32)
    out = _emb(ids, cids, token_table, code_table.reshape(NT * D))
    return out.reshape(B, S, D)


# speedup vs baseline: 2.1550x; 1.6749x over previous
import functools

import jax
import jax.numpy as jnp
from jax import lax
from jax.experimental import pallas as pl
from jax.experimental.pallas import tpu as pltpu
from jax.experimental.pallas import tpu_sc as plsc

B, S = 4, 8192
D = 1024
NT = 8
N = B * S
NC, NS = 2, 16
NW = NC * NS
TOK_PER_W = N // NW
C = 32
NCHUNK = TOK_PER_W // C

_mesh = plsc.VectorSubcoreMesh(core_axis_name="c", subcore_axis_name="s")


@functools.partial(
    pl.kernel,
    mesh=_mesh,
    compiler_params=pltpu.CompilerParams(needs_layout_passes=False),
    out_type=jax.ShapeDtypeStruct((N, D), jnp.float32),
    scratch_types=[
        pltpu.VMEM((TOK_PER_W + C,), jnp.int32),
        pltpu.VMEM((C, D), jnp.float32),
        pltpu.VMEM((C, D), jnp.float32),
        pltpu.SemaphoreType.DMA,
        pltpu.SemaphoreType.DMA,
        pltpu.SemaphoreType.DMA,
        pltpu.SemaphoreType.DMA,
    ],
)
def _gather_sc(ids_hbm, tok_tbl_hbm, out_hbm,
               idx_all, tok0, tok1, gsem0, gsem1, osem0, osem1):
    wid = lax.axis_index("s") * NC + lax.axis_index("c")
    base = wid * TOK_PER_W
    toks = (tok0, tok1)
    gsems = (gsem0, gsem1)
    osems = (osem0, osem1)

    pltpu.sync_copy(ids_hbm.at[pl.ds(base, TOK_PER_W)],
                    idx_all.at[pl.ds(0, TOK_PER_W)])
    for p in range(C // 16):
        idx_all[pl.ds(TOK_PER_W + p * 16, 16)] = jnp.zeros((16,), jnp.int32)

    def start_gather(c, b):
        pltpu.async_copy(tok_tbl_hbm.at[idx_all.at[pl.ds(c * C, C)]],
                         toks[b], gsems[b])

    def wait_gather(b):
        pltpu.make_async_copy(tok_tbl_hbm.at[pl.ds(0, C)], toks[b],
                              gsems[b]).wait()

    def start_out(c, b):
        pltpu.async_copy(toks[b], out_hbm.at[pl.ds(base + c * C, C)],
                         osems[b])

    def wait_out(b):
        pltpu.make_async_copy(toks[b], out_hbm.at[pl.ds(0, C)],
                              osems[b]).wait()

    start_gather(0, 0)

    def _phase(c, carry):
        par = lax.rem(c, 2)
        even = par == 0

        def body(b):
            nb = 1 - b
            @pl.when(c > 0)
            def _():
                wait_out(nb)

            start_gather(c + 1, nb)
            wait_gather(b)
            start_out(c, b)

        @pl.when(even)
        def _():
            body(0)

        @pl.when(jnp.logical_not(even))
        def _():
            body(1)

        return carry

    lax.fori_loop(0, NCHUNK, _phase, 0)

    wait_gather(0)
    wait_out(1)


BLK = 1024


def _add_body(cids_ref, tok_ref, ctbl_ref, out_ref):
    cid = cids_ref[0, 0, :]
    onehot = (cid[:, None] == lax.iota(jnp.int32, NT)[None, :])
    code = jnp.dot(onehot.astype(jnp.float32), ctbl_ref[...],
                   preferred_element_type=jnp.float32)
    out_ref[...] = tok_ref[...] + code


_add_tc = pl.pallas_call(
    _add_body,
    grid=(N // BLK,),
    in_specs=[
        pl.BlockSpec((1, 1, BLK), lambda i: (i, 0, 0)),
        pl.BlockSpec((BLK, D), lambda i: (i, 0)),
        pl.BlockSpec((NT, D), lambda i: (0, 0)),
    ],
    out_specs=pl.BlockSpec((BLK, D), lambda i: (i, 0)),
    out_shape=jax.ShapeDtypeStruct((N, D), jnp.float32),
)


def kernel(input_ids, code_type_ids, token_table, code_table):
    ids = input_ids.reshape(N).astype(jnp.int32)
    cids = code_type_ids.reshape(N // BLK, 1, BLK).astype(jnp.int32)
    gathered = _gather_sc(ids, token_table)
    out = _add_tc(cids, gathered, code_table)
    return out.reshape(B, S, D)
